# baseline (speedup 1.0000x reference)
import jax
import jax.numpy as jnp
from jax import lax
from jax.experimental import pallas as pl
from jax.experimental.pallas import tpu as pltpu

_TOPK = 16
_BK = 2048
_R = _BK // 128
_EXTRA = 64
_PAD_IDX = 2**30


def _dist_block(qb_ref, kt_ref, q2_ref, k2_ref):
    s = lax.dot_general(
        qb_ref[...], kt_ref[...], (((1,), (0,)), ((), ())),
        preferred_element_type=jnp.float32,
    )
    return q2_ref[...] - 2.0 * s + k2_ref[...]


def _tree_min(xs):
    while len(xs) > 1:
        xs = [jnp.minimum(a, b) for a, b in zip(xs[::2], xs[1::2])] + (
            [xs[-1]] if len(xs) % 2 else [])
    return xs[0]


def _pass_a_body(qb_ref, kt_ref, q2_ref, k2_ref, cmin_ref, cidx_ref, thr_ref):
    j = pl.program_id(0)
    nq = qb_ref.shape[0]

    @pl.when(j == 0)
    def _init():
        cmin_ref[...] = jnp.full(cmin_ref.shape, jnp.inf, jnp.float32)
        cidx_ref[...] = jnp.full(cidx_ref.shape, _PAD_IDX, jnp.int32)

    d2 = _dist_block(qb_ref, kt_ref, q2_ref, k2_ref)
    sl = [d2[:, g * 128:(g + 1) * 128] for g in range(_R)]
    bmin = _tree_min(sl)
    barg = jnp.full((nq, 128), _R, jnp.int32)
    for g in reversed(range(_R)):
        barg = jnp.where(sl[g] == bmin, g, barg)
    lane = lax.broadcasted_iota(jnp.int32, (nq, 128), 1)
    bidx = j * _BK + barg * 128 + lane
    upd = bmin < cmin_ref[...]
    cidx_ref[...] = jnp.where(upd, bidx, cidx_ref[...])
    cmin_ref[...] = jnp.where(upd, bmin, cmin_ref[...])

    @pl.when(j == pl.num_programs(0) - 1)
    def _thresh():
        w = cmin_ref[...]
        m = None
        for i in range(_TOPK):
            m = jnp.min(w, axis=1, keepdims=True)
            if i + 1 < _TOPK:
                sp = jnp.min(jnp.where(w == m, lane, _PAD_IDX), axis=1,
                             keepdims=True)
                w = jnp.where(lane == sp, jnp.inf, w)
        thr_ref[...] = m


def _pass_b_body(qb_ref, kt_ref, q2_ref, k2_ref, cmin_ref, cidx_ref, thr_ref,
                 vals_ref, idx_ref, ev_ref, ei_ref, pc_ref, wres_ref):
    j = pl.program_id(0)
    nq = qb_ref.shape[0]

    @pl.when(j == 0)
    def _init():
        ev_ref[...] = jnp.full(ev_ref.shape, jnp.inf, jnp.float32)
        ei_ref[...] = jnp.full(ei_ref.shape, _PAD_IDX, jnp.int32)
        pc_ref[...] = jnp.zeros(pc_ref.shape, jnp.int32)

    d2 = _dist_block(qb_ref, kt_ref, q2_ref, k2_ref)
    t = thr_ref[...]
    cidx = cidx_ref[...]
    lane = lax.broadcasted_iota(jnp.int32, (nq, 128), 1)

    _RT = min(128, nq)
    fmin_p, fsub_p, cl_p = [], [], []
    ln = lax.broadcasted_iota(jnp.int32, (_RT, 128), 1)
    for r in range(0, nq, _RT):
        tr = t[r:r + _RT]
        cr = cidx[r:r + _RT]
        wv_r = []
        clr = jnp.zeros((_RT, 128), jnp.int32)
        for g in range(_R):
            dg = d2[r:r + _RT, g * 128:(g + 1) * 128]
            fl = (dg <= tr) & ((j * _BK + g * 128 + ln) != cr)
            wv_r.append(jnp.where(fl, dg, jnp.inf))
            clr = clr + fl.astype(jnp.int32)
        fm = _tree_min(wv_r)
        fs = jnp.full((_RT, 128), _R, jnp.int32)
        for g in reversed(range(_R)):
            fs = jnp.where(wv_r[g] == fm, g, fs)
        fmin_p.append(fm)
        fsub_p.append(fs)
        cl_p.append(clr)
    fmin = jnp.concatenate(fmin_p, axis=0)
    fsub = jnp.concatenate(fsub_p, axis=0)
    cl = jnp.concatenate(cl_p, axis=0)
    fidx = j * _BK + fsub * 128 + lane
    cn = jnp.sum((fmin < jnp.inf).astype(jnp.int32), axis=1, keepdims=True)
    res = jnp.sum(cl, axis=1, keepdims=True) - cn
    nmax = jnp.max(cn)
    nres = jnp.max(res)
    lane64 = lax.broadcasted_iota(jnp.int32, (nq, _EXTRA), 1)

    def fast(_, fmin):
        m = jnp.min(fmin, axis=1, keepdims=True)
        valid = m < jnp.inf
        si = jnp.min(jnp.where(fmin == m, fidx, _PAD_IDX), axis=1,
                     keepdims=True)
        p = pc_ref[...]
        oh = (lane64 == p) & valid
        ev_ref[...] = jnp.where(oh, m, ev_ref[...])
        ei_ref[...] = jnp.where(oh, si, ei_ref[...])
        pc_ref[...] = p + valid.astype(jnp.int32)
        return jnp.where(fidx == si, jnp.inf, fmin)

    lax.fori_loop(0, nmax, fast, fmin)

    @pl.when(nres > 0)
    def _residuals():
        for g in range(_R):
            dg = d2[:, g * 128:(g + 1) * 128]
            fl = (dg <= t) & ((j * _BK + g * 128 + lane) != cidx)
            wres_ref[:, g * 128:(g + 1) * 128] = jnp.where(
                fl & (fsub != g), dg, jnp.inf)

        def slow(_, __):
            wr = [wres_ref[:, g * 128:(g + 1) * 128] for g in range(_R)]
            fm2 = _tree_min(list(wr))
            m = jnp.min(fm2, axis=1, keepdims=True)
            valid = m < jnp.inf
            gg = jnp.full((nq, 128), _R, jnp.int32)
            for g in reversed(range(_R)):
                gg = jnp.where(wr[g] == fm2, g, gg)
            idx2 = j * _BK + gg * 128 + lane
            si = jnp.min(jnp.where(fm2 == m, idx2, _PAD_IDX), axis=1,
                         keepdims=True)
            p = pc_ref[...]
            oh = (lane64 == p) & valid
            ev_ref[...] = jnp.where(oh, m, ev_ref[...])
            ei_ref[...] = jnp.where(oh, si, ei_ref[...])
            pc_ref[...] = p + valid.astype(jnp.int32)
            for g in range(_R):
                wres_ref[:, g * 128:(g + 1) * 128] = jnp.where(
                    (j * _BK + g * 128 + lane) == si, jnp.inf, wr[g])
            return 0

        lax.fori_loop(0, nres, slow, 0)

    @pl.when(j == pl.num_programs(0) - 1)
    def _merge():
        cv = jnp.concatenate([cmin_ref[...], ev_ref[...]], axis=1)
        ci = jnp.concatenate([cidx_ref[...], ei_ref[...]], axis=1)
        for i in range(_TOPK):
            m = jnp.min(cv, axis=1, keepdims=True)
            si = jnp.min(jnp.where(cv == m, ci, _PAD_IDX), axis=1,
                         keepdims=True)
            vals_ref[:, i:i + 1] = m
            idx_ref[:, i:i + 1] = si
            if i + 1 < _TOPK:
                cv = jnp.where(ci == si, jnp.inf, cv)


def kernel(queries, keys):
    nq, d = queries.shape
    nk = keys.shape[0]
    nkb = (nk + _BK - 1) // _BK
    nkp = nkb * _BK
    q2 = jnp.sum(queries * queries, axis=1, keepdims=True)
    k2 = jnp.concatenate(
        [jnp.sum(keys * keys, axis=1),
         jnp.full((nkp - nk,), jnp.inf, jnp.float32)])[None, :]
    qb = queries.astype(jnp.bfloat16)
    kt = jnp.pad(keys.astype(jnp.bfloat16), ((0, nkp - nk), (0, 0))).T

    const2 = lambda shape: pl.BlockSpec(shape, lambda j: (0, 0))
    stream_specs = [
        const2((nq, d)),
        pl.BlockSpec((d, _BK), lambda j: (0, j)),
        const2((nq, 1)),
        pl.BlockSpec((1, _BK), lambda j: (0, j)),
    ]

    cmin, cidx, thr = pl.pallas_call(
        _pass_a_body,
        grid=(nkb,),
        in_specs=stream_specs,
        out_specs=[const2((nq, 128)), const2((nq, 128)), const2((nq, 1))],
        out_shape=[
            jax.ShapeDtypeStruct((nq, 128), jnp.float32),
            jax.ShapeDtypeStruct((nq, 128), jnp.int32),
            jax.ShapeDtypeStruct((nq, 1), jnp.float32),
        ],
        compiler_params=pltpu.CompilerParams(
            dimension_semantics=("arbitrary",),
        ),
    )(qb, kt, q2, k2)

    vals, idx = pl.pallas_call(
        _pass_b_body,
        grid=(nkb,),
        in_specs=stream_specs + [const2((nq, 128)), const2((nq, 128)),
                                 const2((nq, 1))],
        out_specs=[const2((nq, _TOPK)), const2((nq, _TOPK))],
        out_shape=[
            jax.ShapeDtypeStruct((nq, _TOPK), jnp.float32),
            jax.ShapeDtypeStruct((nq, _TOPK), jnp.int32),
        ],
        scratch_shapes=[
            pltpu.VMEM((nq, _EXTRA), jnp.float32),
            pltpu.VMEM((nq, _EXTRA), jnp.int32),
            pltpu.VMEM((nq, 1), jnp.int32),
            pltpu.VMEM((nq, _BK), jnp.float32),
        ],
        compiler_params=pltpu.CompilerParams(
            dimension_semantics=("arbitrary",),
        ),
    )(qb, kt, q2, k2, cmin, cidx, thr)
    return vals, idx


# speedup vs baseline: 1.0021x; 1.0021x over previous
import jax
import jax.numpy as jnp
from jax import lax
from jax.experimental import pallas as pl
from jax.experimental.pallas import tpu as pltpu

_TOPK = 16
_BK = 2048
_R = _BK // 128
_EXTRA = 64
_PAD_IDX = 2**30


def _dist_block(qb_ref, kt_ref, q2_ref, k2_ref):
    s = lax.dot_general(
        qb_ref[...], kt_ref[...], (((1,), (0,)), ((), ())),
        preferred_element_type=jnp.float32,
    )
    return q2_ref[...] - 2.0 * s + k2_ref[...]


def _tree_min(xs):
    while len(xs) > 1:
        xs = [jnp.minimum(a, b) for a, b in zip(xs[::2], xs[1::2])] + (
            [xs[-1]] if len(xs) % 2 else [])
    return xs[0]


def _pass_a_body(qb_ref, kt_ref, q2_ref, k2_ref, cmin_ref, cidx_ref, thr_ref):
    j = pl.program_id(0)
    nq = qb_ref.shape[0]

    @pl.when(j == 0)
    def _init():
        cmin_ref[...] = jnp.full(cmin_ref.shape, jnp.inf, jnp.float32)
        cidx_ref[...] = jnp.full(cidx_ref.shape, _PAD_IDX, jnp.int32)

    d2 = _dist_block(qb_ref, kt_ref, q2_ref, k2_ref)
    sl = [d2[:, g * 128:(g + 1) * 128] for g in range(_R)]
    bmin = _tree_min(sl)
    barg = jnp.full((nq, 128), _R, jnp.int32)
    for g in reversed(range(_R)):
        barg = jnp.where(sl[g] == bmin, g, barg)
    lane = lax.broadcasted_iota(jnp.int32, (nq, 128), 1)
    bidx = j * _BK + barg * 128 + lane
    upd = bmin < cmin_ref[...]
    cidx_ref[...] = jnp.where(upd, bidx, cidx_ref[...])
    cmin_ref[...] = jnp.where(upd, bmin, cmin_ref[...])

    @pl.when(j == pl.num_programs(0) - 1)
    def _thresh():
        w = cmin_ref[...]
        m = None
        for i in range(_TOPK):
            m = jnp.min(w, axis=1, keepdims=True)
            if i + 1 < _TOPK:
                sp = jnp.min(jnp.where(w == m, lane, _PAD_IDX), axis=1,
                             keepdims=True)
                w = jnp.where(lane == sp, jnp.inf, w)
        thr_ref[...] = m


def _pass_b_body(qb_ref, kt_ref, q2_ref, k2_ref, cmin_ref, cidx_ref, thr_ref,
                 vals_ref, idx_ref, ev_ref, ei_ref, pc_ref, wres_ref):
    j = pl.program_id(0)
    nq = qb_ref.shape[0]

    @pl.when(j == 0)
    def _init():
        ev_ref[...] = jnp.full(ev_ref.shape, jnp.inf, jnp.float32)
        ei_ref[...] = jnp.full(ei_ref.shape, _PAD_IDX, jnp.int32)
        pc_ref[...] = jnp.zeros(pc_ref.shape, jnp.int32)

    d2 = _dist_block(qb_ref, kt_ref, q2_ref, k2_ref)
    t = thr_ref[...]
    cidx = cidx_ref[...]
    lane = lax.broadcasted_iota(jnp.int32, (nq, 128), 1)

    wv = []
    cl = jnp.zeros((nq, 128), jnp.int32)
    for g in range(_R):
        dg = d2[:, g * 128:(g + 1) * 128]
        fl = (dg <= t) & ((j * _BK + g * 128 + lane) != cidx)
        wv.append(jnp.where(fl, dg, jnp.inf))
        cl = cl + fl.astype(jnp.int32)
    fmin = _tree_min(wv)
    fsub = jnp.full((nq, 128), _R, jnp.int32)
    for g in reversed(range(_R)):
        fsub = jnp.where(wv[g] == fmin, g, fsub)
    fidx = j * _BK + fsub * 128 + lane
    cn = jnp.sum((fmin < jnp.inf).astype(jnp.int32), axis=1, keepdims=True)
    res = jnp.sum(cl, axis=1, keepdims=True) - cn
    nmax = jnp.max(cn)
    nres = jnp.max(res)
    lane64 = lax.broadcasted_iota(jnp.int32, (nq, _EXTRA), 1)

    def fast(_, fmin):
        m = jnp.min(fmin, axis=1, keepdims=True)
        valid = m < jnp.inf
        si = jnp.min(jnp.where(fmin == m, fidx, _PAD_IDX), axis=1,
                     keepdims=True)
        p = pc_ref[...]
        oh = (lane64 == p) & valid
        ev_ref[...] = jnp.where(oh, m, ev_ref[...])
        ei_ref[...] = jnp.where(oh, si, ei_ref[...])
        pc_ref[...] = p + valid.astype(jnp.int32)
        return jnp.where(fidx == si, jnp.inf, fmin)

    lax.fori_loop(0, nmax, fast, fmin)

    @pl.when(nres > 0)
    def _residuals():
        for g in range(_R):
            dg = d2[:, g * 128:(g + 1) * 128]
            fl = (dg <= t) & ((j * _BK + g * 128 + lane) != cidx)
            wres_ref[:, g * 128:(g + 1) * 128] = jnp.where(
                fl & (fsub != g), dg, jnp.inf)

        def slow(_, __):
            wr = [wres_ref[:, g * 128:(g + 1) * 128] for g in range(_R)]
            fm2 = _tree_min(list(wr))
            m = jnp.min(fm2, axis=1, keepdims=True)
            valid = m < jnp.inf
            gg = jnp.full((nq, 128), _R, jnp.int32)
            for g in reversed(range(_R)):
                gg = jnp.where(wr[g] == fm2, g, gg)
            idx2 = j * _BK + gg * 128 + lane
            si = jnp.min(jnp.where(fm2 == m, idx2, _PAD_IDX), axis=1,
                         keepdims=True)
            p = pc_ref[...]
            oh = (lane64 == p) & valid
            ev_ref[...] = jnp.where(oh, m, ev_ref[...])
            ei_ref[...] = jnp.where(oh, si, ei_ref[...])
            pc_ref[...] = p + valid.astype(jnp.int32)
            for g in range(_R):
                wres_ref[:, g * 128:(g + 1) * 128] = jnp.where(
                    (j * _BK + g * 128 + lane) == si, jnp.inf, wr[g])
            return 0

        lax.fori_loop(0, nres, slow, 0)

    @pl.when(j == pl.num_programs(0) - 1)
    def _merge():
        cv = jnp.concatenate([cmin_ref[...], ev_ref[...]], axis=1)
        ci = jnp.concatenate([cidx_ref[...], ei_ref[...]], axis=1)
        for i in range(_TOPK):
            m = jnp.min(cv, axis=1, keepdims=True)
            si = jnp.min(jnp.where(cv == m, ci, _PAD_IDX), axis=1,
                         keepdims=True)
            vals_ref[:, i:i + 1] = m
            idx_ref[:, i:i + 1] = si
            if i + 1 < _TOPK:
                cv = jnp.where(ci == si, jnp.inf, cv)


def kernel(queries, keys):
    nq, d = queries.shape
    nk = keys.shape[0]
    nkb = (nk + _BK - 1) // _BK
    nkp = nkb * _BK
    q2 = jnp.sum(queries * queries, axis=1, keepdims=True)
    k2 = jnp.concatenate(
        [jnp.sum(keys * keys, axis=1),
         jnp.full((nkp - nk,), jnp.inf, jnp.float32)])[None, :]
    qb = queries.astype(jnp.bfloat16)
    kt = jnp.pad(keys.astype(jnp.bfloat16), ((0, nkp - nk), (0, 0))).T

    const2 = lambda shape: pl.BlockSpec(shape, lambda j: (0, 0))
    stream_specs = [
        const2((nq, d)),
        pl.BlockSpec((d, _BK), lambda j: (0, j)),
        const2((nq, 1)),
        pl.BlockSpec((1, _BK), lambda j: (0, j)),
    ]

    cmin, cidx, thr = pl.pallas_call(
        _pass_a_body,
        grid=(nkb,),
        in_specs=stream_specs,
        out_specs=[const2((nq, 128)), const2((nq, 128)), const2((nq, 1))],
        out_shape=[
            jax.ShapeDtypeStruct((nq, 128), jnp.float32),
            jax.ShapeDtypeStruct((nq, 128), jnp.int32),
            jax.ShapeDtypeStruct((nq, 1), jnp.float32),
        ],
        compiler_params=pltpu.CompilerParams(
            dimension_semantics=("arbitrary",),
        ),
    )(qb, kt, q2, k2)

    vals, idx = pl.pallas_call(
        _pass_b_body,
        grid=(nkb,),
        in_specs=stream_specs + [const2((nq, 128)), const2((nq, 128)),
                                 const2((nq, 1))],
        out_specs=[const2((nq, _TOPK)), const2((nq, _TOPK))],
        out_shape=[
            jax.ShapeDtypeStruct((nq, _TOPK), jnp.float32),
            jax.ShapeDtypeStruct((nq, _TOPK), jnp.int32),
        ],
        scratch_shapes=[
            pltpu.VMEM((nq, _EXTRA), jnp.float32),
            pltpu.VMEM((nq, _EXTRA), jnp.int32),
            pltpu.VMEM((nq, 1), jnp.int32),
            pltpu.VMEM((nq, _BK), jnp.float32),
        ],
        compiler_params=pltpu.CompilerParams(
            dimension_semantics=("arbitrary",),
        ),
    )(qb, kt, q2, k2, cmin, cidx, thr)
    return vals, idx


# speedup vs baseline: 1.0239x; 1.0218x over previous
import jax
import jax.numpy as jnp
from jax import lax
from jax.experimental import pallas as pl
from jax.experimental.pallas import tpu as pltpu

_TOPK = 16
_BK = 2048
_R = _BK // 128
_EXTRA = 64
_PAD_IDX = 2**30


def _dist_block(qb_ref, kt_ref, q2_ref, k2_ref):
    s = lax.dot_general(
        qb_ref[...], kt_ref[...], (((1,), (0,)), ((), ())),
        preferred_element_type=jnp.float32,
    )
    return q2_ref[...] - 2.0 * s + k2_ref[...]


def _tree_min(xs):
    while len(xs) > 1:
        xs = [jnp.minimum(a, b) for a, b in zip(xs[::2], xs[1::2])] + (
            [xs[-1]] if len(xs) % 2 else [])
    return xs[0]


def _pass_a_body(qb_ref, kt_ref, q2_ref, k2_ref, cmin_ref, cidx_ref, thr_ref):
    j = pl.program_id(0)
    nq = qb_ref.shape[0]

    @pl.when(j == 0)
    def _init():
        cmin_ref[...] = jnp.full(cmin_ref.shape, jnp.inf, jnp.float32)
        cidx_ref[...] = jnp.full(cidx_ref.shape, _PAD_IDX, jnp.int32)

    d2 = _dist_block(qb_ref, kt_ref, q2_ref, k2_ref)
    sl = [d2[:, g * 128:(g + 1) * 128] for g in range(_R)]
    bmin = _tree_min(sl)
    barg = jnp.full((nq, 128), _R, jnp.int32)
    for g in reversed(range(_R)):
        barg = jnp.where(sl[g] == bmin, g, barg)
    lane = lax.broadcasted_iota(jnp.int32, (nq, 128), 1)
    bidx = j * _BK + barg * 128 + lane
    upd = bmin < cmin_ref[...]
    cidx_ref[...] = jnp.where(upd, bidx, cidx_ref[...])
    cmin_ref[...] = jnp.where(upd, bmin, cmin_ref[...])

    @pl.when(j == pl.num_programs(0) - 1)
    def _thresh():
        w = cmin_ref[...]
        m = None
        for i in range(_TOPK):
            m = jnp.min(w, axis=1, keepdims=True)
            if i + 1 < _TOPK:
                sp = jnp.min(jnp.where(w == m, lane, _PAD_IDX), axis=1,
                             keepdims=True)
                w = jnp.where(lane == sp, jnp.inf, w)
        thr_ref[...] = m


def _pass_b_body(qb_ref, kt_ref, q2_ref, k2_ref, cmin_ref, cidx_ref, thr_ref,
                 vals_ref, idx_ref, ev_ref, ei_ref, pc_ref, wres_ref):
    j = pl.program_id(0)
    nq = qb_ref.shape[0]

    @pl.when(j == 0)
    def _init():
        ev_ref[...] = jnp.full(ev_ref.shape, jnp.inf, jnp.float32)
        ei_ref[...] = jnp.full(ei_ref.shape, _PAD_IDX, jnp.int32)
        pc_ref[...] = jnp.zeros(pc_ref.shape, jnp.int32)

    d2 = _dist_block(qb_ref, kt_ref, q2_ref, k2_ref)
    t = thr_ref[...]
    cidx = cidx_ref[...]
    lane = lax.broadcasted_iota(jnp.int32, (nq, 128), 1)

    wv = []
    cl = jnp.zeros((nq, 128), jnp.int32)
    for g in range(_R):
        dg = d2[:, g * 128:(g + 1) * 128]
        fl = (dg <= t) & ((j * _BK + g * 128 + lane) != cidx)
        wv.append(jnp.where(fl, dg, jnp.inf))
        cl = cl + fl.astype(jnp.int32)
    fmin = _tree_min(wv)
    fsub = jnp.full((nq, 128), _R, jnp.int32)
    for g in reversed(range(_R)):
        fsub = jnp.where(wv[g] == fmin, g, fsub)
    fidx = j * _BK + fsub * 128 + lane
    cn = jnp.sum((fmin < jnp.inf).astype(jnp.int32), axis=1, keepdims=True)
    res = jnp.sum(cl, axis=1, keepdims=True) - cn
    nmax = jnp.max(cn)
    nres = jnp.max(res)
    lane64 = lax.broadcasted_iota(jnp.int32, (nq, _EXTRA), 1)

    def fast(_, fmin):
        m = jnp.min(fmin, axis=1, keepdims=True)
        valid = m < jnp.inf
        si = jnp.min(jnp.where(fmin == m, fidx, _PAD_IDX), axis=1,
                     keepdims=True)
        p = pc_ref[...]
        oh = (lane64 == p) & valid
        ev_ref[...] = jnp.where(oh, m, ev_ref[...])
        ei_ref[...] = jnp.where(oh, si, ei_ref[...])
        pc_ref[...] = p + valid.astype(jnp.int32)
        return jnp.where(fidx == si, jnp.inf, fmin)

    lax.fori_loop(0, nmax, fast, fmin)

    @pl.when(nres > 0)
    def _residuals():
        for g in range(_R):
            wres_ref[:, g * 128:(g + 1) * 128] = jnp.where(
                fsub == g, jnp.inf, wv[g])

        def slow(_, __):
            wr = [wres_ref[:, g * 128:(g + 1) * 128] for g in range(_R)]
            fm2 = _tree_min(list(wr))
            m = jnp.min(fm2, axis=1, keepdims=True)
            valid = m < jnp.inf
            gg = jnp.full((nq, 128), _R, jnp.int32)
            for g in reversed(range(_R)):
                gg = jnp.where(wr[g] == fm2, g, gg)
            idx2 = j * _BK + gg * 128 + lane
            si = jnp.min(jnp.where(fm2 == m, idx2, _PAD_IDX), axis=1,
                         keepdims=True)
            p = pc_ref[...]
            oh = (lane64 == p) & valid
            ev_ref[...] = jnp.where(oh, m, ev_ref[...])
            ei_ref[...] = jnp.where(oh, si, ei_ref[...])
            pc_ref[...] = p + valid.astype(jnp.int32)
            for g in range(_R):
                wres_ref[:, g * 128:(g + 1) * 128] = jnp.where(
                    (j * _BK + g * 128 + lane) == si, jnp.inf, wr[g])
            return 0

        lax.fori_loop(0, nres, slow, 0)

    @pl.when(j == pl.num_programs(0) - 1)
    def _merge():
        cv = jnp.concatenate([cmin_ref[...], ev_ref[...]], axis=1)
        ci = jnp.concatenate([cidx_ref[...], ei_ref[...]], axis=1)
        for i in range(_TOPK):
            m = jnp.min(cv, axis=1, keepdims=True)
            si = jnp.min(jnp.where(cv == m, ci, _PAD_IDX), axis=1,
                         keepdims=True)
            vals_ref[:, i:i + 1] = m
            idx_ref[:, i:i + 1] = si
            if i + 1 < _TOPK:
                cv = jnp.where(ci == si, jnp.inf, cv)


def kernel(queries, keys):
    nq, d = queries.shape
    nk = keys.shape[0]
    nkb = (nk + _BK - 1) // _BK
    nkp = nkb * _BK
    q2 = jnp.sum(queries * queries, axis=1, keepdims=True)
    k2 = jnp.concatenate(
        [jnp.sum(keys * keys, axis=1),
         jnp.full((nkp - nk,), jnp.inf, jnp.float32)])[None, :]
    qb = queries.astype(jnp.bfloat16)
    kt = jnp.pad(keys.astype(jnp.bfloat16), ((0, nkp - nk), (0, 0))).T

    const2 = lambda shape: pl.BlockSpec(shape, lambda j: (0, 0))
    stream_specs = [
        const2((nq, d)),
        pl.BlockSpec((d, _BK), lambda j: (0, j)),
        const2((nq, 1)),
        pl.BlockSpec((1, _BK), lambda j: (0, j)),
    ]

    cmin, cidx, thr = pl.pallas_call(
        _pass_a_body,
        grid=(nkb,),
        in_specs=stream_specs,
        out_specs=[const2((nq, 128)), const2((nq, 128)), const2((nq, 1))],
        out_shape=[
            jax.ShapeDtypeStruct((nq, 128), jnp.float32),
            jax.ShapeDtypeStruct((nq, 128), jnp.int32),
            jax.ShapeDtypeStruct((nq, 1), jnp.float32),
        ],
        compiler_params=pltpu.CompilerParams(
            dimension_semantics=("arbitrary",),
        ),
    )(qb, kt, q2, k2)

    vals, idx = pl.pallas_call(
        _pass_b_body,
        grid=(nkb,),
        in_specs=stream_specs + [const2((nq, 128)), const2((nq, 128)),
                                 const2((nq, 1))],
        out_specs=[const2((nq, _TOPK)), const2((nq, _TOPK))],
        out_shape=[
            jax.ShapeDtypeStruct((nq, _TOPK), jnp.float32),
            jax.ShapeDtypeStruct((nq, _TOPK), jnp.int32),
        ],
        scratch_shapes=[
            pltpu.VMEM((nq, _EXTRA), jnp.float32),
            pltpu.VMEM((nq, _EXTRA), jnp.int32),
            pltpu.VMEM((nq, 1), jnp.int32),
            pltpu.VMEM((nq, _BK), jnp.float32),
        ],
        compiler_params=pltpu.CompilerParams(
            dimension_semantics=("arbitrary",),
        ),
    )(qb, kt, q2, k2, cmin, cidx, thr)
    return vals, idx


# speedup vs baseline: 1.0902x; 1.0647x over previous
import jax
import jax.numpy as jnp
from jax import lax
from jax.experimental import pallas as pl
from jax.experimental.pallas import tpu as pltpu

_TOPK = 16
_BK = 2048
_R = _BK // 128
_EXTRA = 64
_PAD_IDX = 2**30


def _dist_block(qb_ref, kt_ref, q2_ref, k2_ref):
    s = lax.dot_general(
        qb_ref[...], kt_ref[...], (((1,), (1,)), ((), ())),
        preferred_element_type=jnp.float32,
    )
    return q2_ref[...] - 2.0 * s + k2_ref[...]


def _tree_min(xs):
    while len(xs) > 1:
        xs = [jnp.minimum(a, b) for a, b in zip(xs[::2], xs[1::2])] + (
            [xs[-1]] if len(xs) % 2 else [])
    return xs[0]


def _pass_a_body(qb_ref, kt_ref, q2_ref, k2_ref, cmin_ref, cidx_ref, thr_ref):
    j = pl.program_id(0)
    nq = qb_ref.shape[0]

    @pl.when(j == 0)
    def _init():
        cmin_ref[...] = jnp.full(cmin_ref.shape, jnp.inf, jnp.float32)
        cidx_ref[...] = jnp.full(cidx_ref.shape, _PAD_IDX, jnp.int32)

    d2 = _dist_block(qb_ref, kt_ref, q2_ref, k2_ref)
    sl = [d2[:, g * 128:(g + 1) * 128] for g in range(_R)]
    bmin = _tree_min(sl)
    barg = jnp.full((nq, 128), _R, jnp.int32)
    for g in reversed(range(_R)):
        barg = jnp.where(sl[g] == bmin, g, barg)
    lane = lax.broadcasted_iota(jnp.int32, (nq, 128), 1)
    bidx = j * _BK + barg * 128 + lane
    upd = bmin < cmin_ref[...]
    cidx_ref[...] = jnp.where(upd, bidx, cidx_ref[...])
    cmin_ref[...] = jnp.where(upd, bmin, cmin_ref[...])

    @pl.when(j == pl.num_programs(0) - 1)
    def _thresh():
        w = cmin_ref[...]
        m = None
        for i in range(_TOPK):
            m = jnp.min(w, axis=1, keepdims=True)
            if i + 1 < _TOPK:
                sp = jnp.min(jnp.where(w == m, lane, _PAD_IDX), axis=1,
                             keepdims=True)
                w = jnp.where(lane == sp, jnp.inf, w)
        thr_ref[...] = m


def _pass_b_body(qb_ref, kt_ref, q2_ref, k2_ref, cmin_ref, cidx_ref, thr_ref,
                 vals_ref, idx_ref, ev_ref, ei_ref, pc_ref, wres_ref):
    j = pl.program_id(0)
    nq = qb_ref.shape[0]

    @pl.when(j == 0)
    def _init():
        ev_ref[...] = jnp.full(ev_ref.shape, jnp.inf, jnp.float32)
        ei_ref[...] = jnp.full(ei_ref.shape, _PAD_IDX, jnp.int32)
        pc_ref[...] = jnp.zeros(pc_ref.shape, jnp.int32)

    d2 = _dist_block(qb_ref, kt_ref, q2_ref, k2_ref)
    t = thr_ref[...]
    cidx = cidx_ref[...]
    lane = lax.broadcasted_iota(jnp.int32, (nq, 128), 1)

    wv = []
    cl = jnp.zeros((nq, 128), jnp.int32)
    for g in range(_R):
        dg = d2[:, g * 128:(g + 1) * 128]
        fl = (dg <= t) & ((j * _BK + g * 128 + lane) != cidx)
        wv.append(jnp.where(fl, dg, jnp.inf))
        cl = cl + fl.astype(jnp.int32)
    fmin = _tree_min(wv)
    fsub = jnp.full((nq, 128), _R, jnp.int32)
    for g in reversed(range(_R)):
        fsub = jnp.where(wv[g] == fmin, g, fsub)
    fidx = j * _BK + fsub * 128 + lane
    cn = jnp.sum((fmin < jnp.inf).astype(jnp.int32), axis=1, keepdims=True)
    res = jnp.sum(cl, axis=1, keepdims=True) - cn
    nmax = jnp.max(cn)
    nres = jnp.max(res)
    lane64 = lax.broadcasted_iota(jnp.int32, (nq, _EXTRA), 1)

    def fast(_, fmin):
        m = jnp.min(fmin, axis=1, keepdims=True)
        valid = m < jnp.inf
        si = jnp.min(jnp.where(fmin == m, fidx, _PAD_IDX), axis=1,
                     keepdims=True)
        p = pc_ref[...]
        oh = (lane64 == p) & valid
        ev_ref[...] = jnp.where(oh, m, ev_ref[...])
        ei_ref[...] = jnp.where(oh, si, ei_ref[...])
        pc_ref[...] = p + valid.astype(jnp.int32)
        return jnp.where(fidx == si, jnp.inf, fmin)

    lax.fori_loop(0, nmax, fast, fmin)

    @pl.when(nres > 0)
    def _residuals():
        for g in range(_R):
            wres_ref[:, g * 128:(g + 1) * 128] = jnp.where(
                fsub == g, jnp.inf, wv[g])

        def slow(_, __):
            wr = [wres_ref[:, g * 128:(g + 1) * 128] for g in range(_R)]
            fm2 = _tree_min(list(wr))
            m = jnp.min(fm2, axis=1, keepdims=True)
            valid = m < jnp.inf
            gg = jnp.full((nq, 128), _R, jnp.int32)
            for g in reversed(range(_R)):
                gg = jnp.where(wr[g] == fm2, g, gg)
            idx2 = j * _BK + gg * 128 + lane
            si = jnp.min(jnp.where(fm2 == m, idx2, _PAD_IDX), axis=1,
                         keepdims=True)
            p = pc_ref[...]
            oh = (lane64 == p) & valid
            ev_ref[...] = jnp.where(oh, m, ev_ref[...])
            ei_ref[...] = jnp.where(oh, si, ei_ref[...])
            pc_ref[...] = p + valid.astype(jnp.int32)
            for g in range(_R):
                wres_ref[:, g * 128:(g + 1) * 128] = jnp.where(
                    (j * _BK + g * 128 + lane) == si, jnp.inf, wr[g])
            return 0

        lax.fori_loop(0, nres, slow, 0)

    @pl.when(j == pl.num_programs(0) - 1)
    def _merge():
        cv = jnp.concatenate([cmin_ref[...], ev_ref[...]], axis=1)
        ci = jnp.concatenate([cidx_ref[...], ei_ref[...]], axis=1)
        for i in range(_TOPK):
            m = jnp.min(cv, axis=1, keepdims=True)
            si = jnp.min(jnp.where(cv == m, ci, _PAD_IDX), axis=1,
                         keepdims=True)
            vals_ref[:, i:i + 1] = m
            idx_ref[:, i:i + 1] = si
            if i + 1 < _TOPK:
                cv = jnp.where(ci == si, jnp.inf, cv)


def kernel(queries, keys):
    nq, d = queries.shape
    nk = keys.shape[0]
    nkb = (nk + _BK - 1) // _BK
    nkp = nkb * _BK
    q2 = jnp.sum(queries * queries, axis=1, keepdims=True)
    k2 = jnp.concatenate(
        [jnp.sum(keys * keys, axis=1),
         jnp.full((nkp - nk,), jnp.inf, jnp.float32)])[None, :]
    qb = queries.astype(jnp.bfloat16)
    kt = jnp.pad(keys.astype(jnp.bfloat16), ((0, nkp - nk), (0, 0)))

    const2 = lambda shape: pl.BlockSpec(shape, lambda j: (0, 0))
    stream_specs = [
        const2((nq, d)),
        pl.BlockSpec((_BK, d), lambda j: (j, 0)),
        const2((nq, 1)),
        pl.BlockSpec((1, _BK), lambda j: (0, j)),
    ]

    cmin, cidx, thr = pl.pallas_call(
        _pass_a_body,
        grid=(nkb,),
        in_specs=stream_specs,
        out_specs=[const2((nq, 128)), const2((nq, 128)), const2((nq, 1))],
        out_shape=[
            jax.ShapeDtypeStruct((nq, 128), jnp.float32),
            jax.ShapeDtypeStruct((nq, 128), jnp.int32),
            jax.ShapeDtypeStruct((nq, 1), jnp.float32),
        ],
        compiler_params=pltpu.CompilerParams(
            dimension_semantics=("arbitrary",),
        ),
    )(qb, kt, q2, k2)

    vals, idx = pl.pallas_call(
        _pass_b_body,
        grid=(nkb,),
        in_specs=stream_specs + [const2((nq, 128)), const2((nq, 128)),
                                 const2((nq, 1))],
        out_specs=[const2((nq, _TOPK)), const2((nq, _TOPK))],
        out_shape=[
            jax.ShapeDtypeStruct((nq, _TOPK), jnp.float32),
            jax.ShapeDtypeStruct((nq, _TOPK), jnp.int32),
        ],
        scratch_shapes=[
            pltpu.VMEM((nq, _EXTRA), jnp.float32),
            pltpu.VMEM((nq, _EXTRA), jnp.int32),
            pltpu.VMEM((nq, 1), jnp.int32),
            pltpu.VMEM((nq, _BK), jnp.float32),
        ],
        compiler_params=pltpu.CompilerParams(
            dimension_semantics=("arbitrary",),
        ),
    )(qb, kt, q2, k2, cmin, cidx, thr)
    return vals, idx
